# Initial kernel scaffold; baseline (speedup 1.0000x reference)
import jax
import jax.numpy as jnp
from jax.experimental import pallas as pl


def kernel(input_ids, word_emb, rel_emb, rel_gamma, rel_beta):
    raise NotImplementedError("write your pallas kernel here")


# speedup vs baseline: 17.0447x; 17.0447x over previous
import functools

import jax
import jax.numpy as jnp
from jax.experimental import pallas as pl
from jax.experimental.pallas import tpu as pltpu


def _round_up(x, m):
    return ((x + m - 1) // m) * m


def _gather_ln_kernel(ids_sref, table_ref, out_ref, tile_ref, *, tq, p, stride, eps):
    t = pl.program_id(0)
    base = t * tq

    for mi in range(tq):
        idx = pl.multiple_of(ids_sref[base + mi], p)
        slab = table_ref[pl.ds(idx, p), :]
        tile_ref[mi : mi + p * stride : stride, :] = slab

    x = jnp.concatenate(
        [tile_ref[pl.ds(j * stride, tq), :] for j in range(p)], axis=1
    )

    mean = jnp.mean(x, axis=-1, keepdims=True)
    centered = x - mean
    var = jnp.mean(centered * centered, axis=-1, keepdims=True)
    out_ref[...] = centered * jax.lax.rsqrt(var + eps)


def _word_embed_ln(input_ids, emb_table, *, eps, tq=256):
    B, S = input_ids.shape
    V, H = emb_table.shape
    N = B * S
    assert H % 128 == 0, "hidden size must be lane-tile aligned"
    p = H // 128

    tq_eff = min(tq, _round_up(N, 8))
    n_pad = _round_up(N, tq_eff)
    stride = tq_eff + 1

    table2d = emb_table.reshape(V * p, 128)

    ids = jnp.clip(input_ids.reshape(N).astype(jnp.int32), 0, V - 1) * p
    if n_pad != N:
        ids = jnp.pad(ids, (0, n_pad - N))

    table_bytes = V * H * 4
    vmem_limit = min(
        2 * table_bytes + 4 * tq_eff * H * 4 + (8 << 20),
        60 << 20,
    )

    grid_spec = pltpu.PrefetchScalarGridSpec(
        num_scalar_prefetch=1,
        grid=(n_pad // tq_eff,),
        in_specs=[
            pl.BlockSpec((V * p, 128), lambda i, ids_ref: (0, 0)),
        ],
        out_specs=pl.BlockSpec((tq_eff, H), lambda i, ids_ref: (i, 0)),
        scratch_shapes=[pltpu.VMEM((stride * p, 128), jnp.float32)],
    )

    out = pl.pallas_call(
        functools.partial(
            _gather_ln_kernel, tq=tq_eff, p=p, stride=stride, eps=eps
        ),
        out_shape=jax.ShapeDtypeStruct((n_pad, H), jnp.float32),
        grid_spec=grid_spec,
        compiler_params=pltpu.CompilerParams(
            dimension_semantics=("parallel",),
            vmem_limit_bytes=vmem_limit,
        ),
    )(ids, table2d)
    return out[:N].reshape(B, S, H)


def _rel_ln_kernel(x_ref, g_ref, b_ref, out_ref, *, eps):
    x = x_ref[...]
    mean = jnp.mean(x, axis=-1, keepdims=True)
    centered = x - mean
    var = jnp.mean(centered * centered, axis=-1, keepdims=True)
    out_ref[...] = centered * jax.lax.rsqrt(var + eps) * g_ref[...] + b_ref[...]


def _rel_ln(rel_emb, gamma, beta, *, eps):
    R, H = rel_emb.shape
    br = _round_up(_round_up(R, 2) // 2, 8)
    grid = _round_up(R, br) // br
    return pl.pallas_call(
        functools.partial(_rel_ln_kernel, eps=eps),
        out_shape=jax.ShapeDtypeStruct((R, H), rel_emb.dtype),
        grid=(grid,),
        in_specs=[
            pl.BlockSpec((br, H), lambda i: (i, 0)),
            pl.BlockSpec((1, H), lambda i: (0, 0)),
            pl.BlockSpec((1, H), lambda i: (0, 0)),
        ],
        out_specs=pl.BlockSpec((br, H), lambda i: (i, 0)),
        compiler_params=pltpu.CompilerParams(
            dimension_semantics=("parallel",),
        ),
    )(rel_emb, gamma.reshape(1, H), beta.reshape(1, H))


def kernel(input_ids, word_emb, rel_emb, rel_gamma, rel_beta):
    eps = 1e-7
    word = _word_embed_ln(input_ids, word_emb, eps=eps)
    rel = _rel_ln(rel_emb, rel_gamma, rel_beta, eps=eps)
    return word, rel


# speedup vs baseline: 21.9940x; 1.2904x over previous
import functools

import jax
import jax.numpy as jnp
from jax.experimental import pallas as pl
from jax.experimental.pallas import tpu as pltpu


def _round_up(x, m):
    return ((x + m - 1) // m) * m


def _gather_ln_kernel(ids_sref, table_ref, out_ref, tile_ref, *, tq, p, stride, eps):
    t = pl.program_id(0)
    base = t * tq

    for mi in range(tq):
        idx = pl.multiple_of(ids_sref[base + mi], p)
        slab = table_ref[pl.ds(idx, p), :]
        tile_ref[mi : mi + p * stride : stride, :] = slab

    x = jnp.concatenate(
        [tile_ref[pl.ds(j * stride, tq), :] for j in range(p)], axis=1
    )

    mean = jnp.mean(x, axis=-1, keepdims=True)
    centered = x - mean
    var = jnp.mean(centered * centered, axis=-1, keepdims=True)
    out_ref[...] = centered * jax.lax.rsqrt(var + eps)


def _word_embed_ln(input_ids, emb_table, *, eps, tq=512):
    B, S = input_ids.shape
    V, H = emb_table.shape
    N = B * S
    assert H % 128 == 0, "hidden size must be lane-tile aligned"
    p = H // 128

    tq_eff = min(tq, _round_up(N, 8))
    n_pad = _round_up(N, tq_eff)
    stride = tq_eff + 1

    table2d = emb_table.reshape(V * p, 128)

    ids = jnp.clip(input_ids.reshape(N).astype(jnp.int32), 0, V - 1) * p
    if n_pad != N:
        ids = jnp.pad(ids, (0, n_pad - N))

    table_bytes = V * H * 4
    vmem_limit = min(
        2 * table_bytes + 4 * tq_eff * H * 4 + (8 << 20),
        60 << 20,
    )

    grid_spec = pltpu.PrefetchScalarGridSpec(
        num_scalar_prefetch=1,
        grid=(n_pad // tq_eff,),
        in_specs=[
            pl.BlockSpec((V * p, 128), lambda i, ids_ref: (0, 0)),
        ],
        out_specs=pl.BlockSpec((tq_eff, H), lambda i, ids_ref: (i, 0)),
        scratch_shapes=[pltpu.VMEM((stride * p, 128), jnp.float32)],
    )

    out = pl.pallas_call(
        functools.partial(
            _gather_ln_kernel, tq=tq_eff, p=p, stride=stride, eps=eps
        ),
        out_shape=jax.ShapeDtypeStruct((n_pad, H), jnp.float32),
        grid_spec=grid_spec,
        compiler_params=pltpu.CompilerParams(
            dimension_semantics=("parallel",),
            vmem_limit_bytes=vmem_limit,
        ),
    )(ids, table2d)
    return out[:N].reshape(B, S, H)


def _rel_ln_kernel(x_ref, g_ref, b_ref, out_ref, *, eps):
    x = x_ref[...]
    mean = jnp.mean(x, axis=-1, keepdims=True)
    centered = x - mean
    var = jnp.mean(centered * centered, axis=-1, keepdims=True)
    out_ref[...] = centered * jax.lax.rsqrt(var + eps) * g_ref[...] + b_ref[...]


def _rel_ln(rel_emb, gamma, beta, *, eps):
    R, H = rel_emb.shape
    br = _round_up(_round_up(R, 2) // 2, 8)
    grid = _round_up(R, br) // br
    return pl.pallas_call(
        functools.partial(_rel_ln_kernel, eps=eps),
        out_shape=jax.ShapeDtypeStruct((R, H), rel_emb.dtype),
        grid=(grid,),
        in_specs=[
            pl.BlockSpec((br, H), lambda i: (i, 0)),
            pl.BlockSpec((1, H), lambda i: (0, 0)),
            pl.BlockSpec((1, H), lambda i: (0, 0)),
        ],
        out_specs=pl.BlockSpec((br, H), lambda i: (i, 0)),
        compiler_params=pltpu.CompilerParams(
            dimension_semantics=("parallel",),
        ),
    )(rel_emb, gamma.reshape(1, H), beta.reshape(1, H))


def kernel(input_ids, word_emb, rel_emb, rel_gamma, rel_beta):
    eps = 1e-7
    word = _word_embed_ln(input_ids, word_emb, eps=eps)
    rel = _rel_ln(rel_emb, rel_gamma, rel_beta, eps=eps)
    return word, rel


# speedup vs baseline: 24.2278x; 1.1016x over previous
import functools

import jax
import jax.numpy as jnp
from jax.experimental import pallas as pl
from jax.experimental.pallas import tpu as pltpu


def _round_up(x, m):
    return ((x + m - 1) // m) * m


def _gather_ln_kernel(ids_sref, table_ref, out_ref, tile_ref, *, tq, p, stride, eps):
    t = pl.program_id(0)
    base = t * tq

    for mi in range(tq):
        idx = pl.multiple_of(ids_sref[base + mi], p)
        slab = table_ref[pl.ds(idx, p), :]
        tile_ref[mi : mi + p * stride : stride, :] = slab

    x = jnp.concatenate(
        [tile_ref[pl.ds(j * stride, tq), :] for j in range(p)], axis=1
    )

    mean = jnp.mean(x, axis=-1, keepdims=True)
    centered = x - mean
    var = jnp.mean(centered * centered, axis=-1, keepdims=True)
    out_ref[...] = centered * jax.lax.rsqrt(var + eps)


def _word_embed_ln(input_ids, emb_table, *, eps, tq=1024):
    B, S = input_ids.shape
    V, H = emb_table.shape
    N = B * S
    assert H % 128 == 0, "hidden size must be lane-tile aligned"
    p = H // 128

    tq_eff = min(tq, _round_up(N, 8))
    n_pad = _round_up(N, tq_eff)
    stride = tq_eff + 1

    table2d = emb_table.reshape(V * p, 128)

    ids = jnp.clip(input_ids.reshape(N).astype(jnp.int32), 0, V - 1) * p
    if n_pad != N:
        ids = jnp.pad(ids, (0, n_pad - N))

    table_bytes = V * H * 4
    vmem_limit = min(
        2 * table_bytes + 4 * tq_eff * H * 4 + (8 << 20),
        60 << 20,
    )

    grid_spec = pltpu.PrefetchScalarGridSpec(
        num_scalar_prefetch=1,
        grid=(n_pad // tq_eff,),
        in_specs=[
            pl.BlockSpec((V * p, 128), lambda i, ids_ref: (0, 0)),
        ],
        out_specs=pl.BlockSpec((tq_eff, H), lambda i, ids_ref: (i, 0)),
        scratch_shapes=[pltpu.VMEM((stride * p, 128), jnp.float32)],
    )

    out = pl.pallas_call(
        functools.partial(
            _gather_ln_kernel, tq=tq_eff, p=p, stride=stride, eps=eps
        ),
        out_shape=jax.ShapeDtypeStruct((n_pad, H), jnp.float32),
        grid_spec=grid_spec,
        compiler_params=pltpu.CompilerParams(
            dimension_semantics=("parallel",),
            vmem_limit_bytes=vmem_limit,
        ),
    )(ids, table2d)
    return out[:N].reshape(B, S, H)


def _rel_ln_kernel(x_ref, g_ref, b_ref, out_ref, *, eps):
    x = x_ref[...]
    mean = jnp.mean(x, axis=-1, keepdims=True)
    centered = x - mean
    var = jnp.mean(centered * centered, axis=-1, keepdims=True)
    out_ref[...] = centered * jax.lax.rsqrt(var + eps) * g_ref[...] + b_ref[...]


def _rel_ln(rel_emb, gamma, beta, *, eps):
    R, H = rel_emb.shape
    br = _round_up(_round_up(R, 2) // 2, 8)
    grid = _round_up(R, br) // br
    return pl.pallas_call(
        functools.partial(_rel_ln_kernel, eps=eps),
        out_shape=jax.ShapeDtypeStruct((R, H), rel_emb.dtype),
        grid=(grid,),
        in_specs=[
            pl.BlockSpec((br, H), lambda i: (i, 0)),
            pl.BlockSpec((1, H), lambda i: (0, 0)),
            pl.BlockSpec((1, H), lambda i: (0, 0)),
        ],
        out_specs=pl.BlockSpec((br, H), lambda i: (i, 0)),
        compiler_params=pltpu.CompilerParams(
            dimension_semantics=("parallel",),
        ),
    )(rel_emb, gamma.reshape(1, H), beta.reshape(1, H))


def kernel(input_ids, word_emb, rel_emb, rel_gamma, rel_beta):
    eps = 1e-7
    word = _word_embed_ln(input_ids, word_emb, eps=eps)
    rel = _rel_ln(rel_emb, rel_gamma, rel_beta, eps=eps)
    return word, rel
